# Initial kernel scaffold; baseline (speedup 1.0000x reference)
import jax
import jax.numpy as jnp
from jax.experimental import pallas as pl


def kernel(x, edge_index, edge_attr, batch, W_neigh0, b_neigh0, W_root0, b_root0, ln_g0, ln_b0, W_neigh1, b_neigh1, W_root1, b_root1, ln_g1, ln_b1, W_proj, b_proj, w_score, b_score):
    raise NotImplementedError("write your pallas kernel here")


# speedup vs baseline: 4.8032x; 4.8032x over previous
import functools

import jax
import jax.numpy as jnp
from jax import lax
from jax.experimental import pallas as pl
from jax.experimental.pallas import tpu as pltpu
from jax.experimental.pallas import tpu_sc as plsc

N_NODES = 10000
N_EDGES = 320000
D_IN = 128
D_HID = 128
D_EDGE = 16
NUM_GRAPHS = 8

NC = 2
NS = 16
NT = NC * NS
EPT = N_EDGES // NT
K = 80
CH = EPT // K
RPT = 624
TAIL = N_NODES - NS * RPT
DEA = 128

_f32 = jnp.float32


@functools.cache
def _get_mesh():
    return plsc.VectorSubcoreMesh(core_axis_name="c", subcore_axis_name="s",
                                  num_cores=NC, num_subcores=NS)


def _sc_gather_sum_body(h_hbm, srcr_hbm, dstr_hbm, zg_hbm, outg_hbm,
                        src_v, dst_v, rows_v, accg, sem):
    c = lax.axis_index("c")
    s = lax.axis_index("s")
    wid = c * NS + s
    pltpu.sync_copy(zg_hbm.at[pl.ds(0, RPT)], accg.at[pl.ds(s * RPT, RPT)])

    @pl.when(s == NS - 1)
    def _():
        pltpu.sync_copy(zg_hbm.at[pl.ds(0, TAIL)],
                        accg.at[pl.ds(NS * RPT, TAIL)])

    pltpu.sync_copy(srcr_hbm.at[wid], src_v)
    pltpu.sync_copy(dstr_hbm.at[wid], dst_v)
    plsc.subcore_barrier()

    def body(j, carry):
        pltpu.async_copy(h_hbm.at[src_v.at[j]], rows_v, sem).wait()
        pltpu.sync_copy(rows_v, accg.at[dst_v.at[j]], add=True)
        return carry

    lax.fori_loop(0, CH, body, 0)
    plsc.subcore_barrier()
    pltpu.sync_copy(accg.at[pl.ds(s * RPT, RPT)],
                    outg_hbm.at[c, pl.ds(s * RPT, RPT)])

    @pl.when(s == NS - 1)
    def _():
        pltpu.sync_copy(accg.at[pl.ds(NS * RPT, TAIL)],
                        outg_hbm.at[c, pl.ds(NS * RPT, TAIL)])


@functools.cache
def _build_sc_gather_sum():
    return pl.kernel(
        _sc_gather_sum_body,
        out_type=jax.ShapeDtypeStruct((NC, N_NODES, D_HID), _f32),
        mesh=_get_mesh(),
        scratch_types=[
            pltpu.VMEM((CH, K), jnp.int32),
            pltpu.VMEM((CH, K), jnp.int32),
            pltpu.VMEM((K, D_HID), _f32),
            pltpu.VMEM_SHARED((N_NODES, D_HID), _f32),
            pltpu.SemaphoreType.DMA,
        ],
    )


def _sc_edge_sum_body(ea_hbm, dstr_hbm, ze_hbm, oute_hbm,
                      dst_v, ea_v, acce, sem):
    c = lax.axis_index("c")
    s = lax.axis_index("s")
    wid = c * NS + s
    pltpu.sync_copy(ze_hbm.at[pl.ds(0, RPT)], acce.at[pl.ds(s * RPT, RPT)])

    @pl.when(s == NS - 1)
    def _():
        pltpu.sync_copy(ze_hbm.at[pl.ds(0, TAIL)],
                        acce.at[pl.ds(NS * RPT, TAIL)])

    pltpu.sync_copy(dstr_hbm.at[wid], dst_v)
    plsc.subcore_barrier()

    def body(j, carry):
        pltpu.sync_copy(ea_hbm.at[pl.ds(wid * EPT + j * K, K)], ea_v)
        pltpu.sync_copy(ea_v, acce.at[dst_v.at[j]], add=True)
        return carry

    lax.fori_loop(0, CH, body, 0)
    plsc.subcore_barrier()
    pltpu.sync_copy(acce.at[pl.ds(s * RPT, RPT)],
                    oute_hbm.at[c, pl.ds(s * RPT, RPT)])

    @pl.when(s == NS - 1)
    def _():
        pltpu.sync_copy(acce.at[pl.ds(NS * RPT, TAIL)],
                        oute_hbm.at[c, pl.ds(NS * RPT, TAIL)])


@functools.cache
def _build_sc_edge_sum():
    return pl.kernel(
        _sc_edge_sum_body,
        out_type=jax.ShapeDtypeStruct((NC, N_NODES, DEA), _f32),
        mesh=_get_mesh(),
        scratch_types=[
            pltpu.VMEM((CH, K), jnp.int32),
            pltpu.VMEM((K, DEA), _f32),
            pltpu.VMEM_SHARED((N_NODES, DEA), _f32),
            pltpu.SemaphoreType.DMA,
        ],
    )


def _dot(a, b):
    return lax.dot_general(a, b, (((1,), (0,)), ((), ())),
                           precision=lax.Precision.HIGHEST,
                           preferred_element_type=_f32)


def _tc_layer_kernel(gp_ref, ep_ref, hin_ref, wx_ref, we_ref, bn_ref,
                     wr_ref, br_ref, g_ref, b_ref, out_ref):
    gsum = gp_ref[0] + gp_ref[1]
    esum = ep_ref[0] + ep_ref[1]
    cnt = esum[:, D_EDGE:D_EDGE + 1]
    msum = (_dot(gsum, wx_ref[0]) + _dot(esum[:, :D_EDGE], we_ref[0])
            + cnt * bn_ref[...])
    agg = msum / jnp.maximum(cnt, 1.0)
    h = agg + _dot(hin_ref[...], wr_ref[0]) + br_ref[...]
    h = jnp.maximum(h, 0.0)
    mu = jnp.mean(h, axis=-1, keepdims=True)
    var = jnp.mean((h - mu) ** 2, axis=-1, keepdims=True)
    out_ref[...] = (h - mu) * lax.rsqrt(var + 1e-5) * g_ref[...] + b_ref[...]


def _tc_pool_kernel(h_ref, wp_ref, bp_ref, ws_ref, bs_ref, batch_ref,
                    out_ref):
    h = h_ref[...]
    hp = jnp.tanh(_dot(h, wp_ref[0]) + bp_ref[...])
    s = _dot(hp, ws_ref[...]) + bs_ref[...]
    gid = lax.broadcasted_iota(jnp.int32, (N_NODES, NUM_GRAPHS), 1)
    oh = (batch_ref[...] == gid)
    ohf = oh.astype(_f32)
    smax = jnp.max(jnp.where(oh, s, -1e30), axis=0, keepdims=True)
    smax_b = jnp.sum(jnp.where(oh, smax, 0.0), axis=1, keepdims=True)
    e = jnp.exp(s - smax_b)
    denom = jnp.sum(ohf * e, axis=0, keepdims=True)
    denom_b = jnp.sum(ohf * denom, axis=1, keepdims=True)
    w = e / denom_b
    out_ref[...] = lax.dot_general(
        ohf * w, h, (((0,), (0,)), ((), ())),
        precision=lax.Precision.HIGHEST, preferred_element_type=_f32)


_RB = 1000
_NRB = N_NODES // _RB


def _tc_layer(gp, ep, hin, wn, bn, wr, br, g, b):
    return pl.pallas_call(
        _tc_layer_kernel,
        grid=(_NRB,),
        in_specs=[
            pl.BlockSpec((NC, _RB, D_HID), lambda i: (0, i, 0)),
            pl.BlockSpec((NC, _RB, DEA), lambda i: (0, i, 0)),
            pl.BlockSpec((_RB, D_HID), lambda i: (i, 0)),
            pl.BlockSpec((1, D_HID, D_HID), lambda i: (0, 0, 0)),
            pl.BlockSpec((1, D_EDGE, D_HID), lambda i: (0, 0, 0)),
            pl.BlockSpec((1, D_HID), lambda i: (0, 0)),
            pl.BlockSpec((1, D_HID, D_HID), lambda i: (0, 0, 0)),
            pl.BlockSpec((1, D_HID), lambda i: (0, 0)),
            pl.BlockSpec((1, D_HID), lambda i: (0, 0)),
            pl.BlockSpec((1, D_HID), lambda i: (0, 0)),
        ],
        out_specs=pl.BlockSpec((_RB, D_HID), lambda i: (i, 0)),
        out_shape=jax.ShapeDtypeStruct((N_NODES, D_HID), _f32),
    )(gp, ep, hin,
      wn[:D_HID].reshape(1, D_HID, D_HID),
      wn[D_HID:].reshape(1, D_EDGE, D_HID),
      bn.reshape(1, D_HID),
      wr.reshape(1, D_HID, D_HID),
      br.reshape(1, D_HID),
      g.reshape(1, D_HID), b.reshape(1, D_HID))


def kernel(x, edge_index, edge_attr, batch,
           W_neigh0, b_neigh0, W_root0, b_root0, ln_g0, ln_b0,
           W_neigh1, b_neigh1, W_root1, b_root1, ln_g1, ln_b1,
           W_proj, b_proj, w_score, b_score):
    src_r = edge_index[0].reshape(NT, CH, K)
    dst_r = edge_index[1].reshape(NT, CH, K)
    ea_pad = jnp.concatenate(
        [edge_attr,
         jnp.ones((N_EDGES, 1), _f32),
         jnp.zeros((N_EDGES, DEA - D_EDGE - 1), _f32)], axis=1)
    zg = jnp.zeros((RPT, D_HID), _f32)
    ze = jnp.zeros((RPT, DEA), _f32)

    gather_sum = _build_sc_gather_sum()
    ep = _build_sc_edge_sum()(ea_pad, dst_r, ze)
    gp0 = gather_sum(x, src_r, dst_r, zg)
    h0 = _tc_layer(gp0, ep, x, W_neigh0, b_neigh0, W_root0, b_root0,
                   ln_g0, ln_b0)
    gp1 = gather_sum(h0, src_r, dst_r, zg)
    h1 = _tc_layer(gp1, ep, h0, W_neigh1, b_neigh1, W_root1, b_root1,
                   ln_g1, ln_b1)

    pooled = pl.pallas_call(
        _tc_pool_kernel,
        out_shape=jax.ShapeDtypeStruct((NUM_GRAPHS, D_HID), _f32),
    )(h1, W_proj.reshape(1, D_HID, D_HID), b_proj.reshape(1, D_HID),
      w_score.reshape(D_HID, 1), b_score.reshape(1, 1),
      batch.reshape(N_NODES, 1))
    return pooled
